# baseline (speedup 1.0000x reference)
import functools

import jax
import jax.numpy as jnp
from jax import lax
from jax.experimental import pallas as pl
from jax.experimental.pallas import tpu as pltpu
from jax.experimental.pallas import tpu_sc as plsc

N_EMB = 1024
D_EMB = 256
COMMIT_BETA = 0.25
TOTAL_M = 16 * 32 * 32
BLOCK_M = 512
SC_CHUNK = 256


def _tc_distance_argmin_body(x_ref, e_ref, idx_ref, loss_ref):
    i = pl.program_id(0)
    x = x_ref[...]
    e = e_ref[...]
    rs = jnp.sum(x * x, axis=1, keepdims=True)
    c = jnp.sum(e * e, axis=0, keepdims=True)
    mm = lax.dot_general(x, e, (((1,), (0,)), ((), ())),
                         preferred_element_type=jnp.float32)
    d = (rs + c) - 2.0 * mm
    dmin = jnp.min(d, axis=1, keepdims=True)
    iota = lax.broadcasted_iota(jnp.int32, d.shape, 1)
    idx_ref[...] = jnp.min(jnp.where(d == dmin, iota, N_EMB),
                           axis=1, keepdims=True)
    part = jnp.sum(dmin).reshape(1, 1)

    @pl.when(i == 0)
    def _init():
        loss_ref[...] = part

    @pl.when(i > 0)
    def _acc():
        loss_ref[...] += part


def _tc_distance_argmin(flat, embeddings):
    return pl.pallas_call(
        _tc_distance_argmin_body,
        grid=(TOTAL_M // BLOCK_M,),
        in_specs=[
            pl.BlockSpec((BLOCK_M, D_EMB), lambda i: (i, 0)),
            pl.BlockSpec((D_EMB, N_EMB), lambda i: (0, 0)),
        ],
        out_specs=[
            pl.BlockSpec((BLOCK_M, 1), lambda i: (i, 0)),
            pl.BlockSpec((1, 1), lambda i: (0, 0)),
        ],
        out_shape=[
            jax.ShapeDtypeStruct((TOTAL_M, 1), jnp.int32),
            jax.ShapeDtypeStruct((1, 1), jnp.float32),
        ],
    )(flat, embeddings)


def _sc_gather_rows(table, idx_flat):
    info = plsc.get_sparse_core_info()
    n_workers = info.num_cores * info.num_subcores
    b_per_w = TOTAL_M // n_workers
    n_chunks = b_per_w // SC_CHUNK
    mesh = plsc.VectorSubcoreMesh(core_axis_name="c", subcore_axis_name="s")

    @functools.partial(
        pl.kernel, mesh=mesh,
        out_type=jax.ShapeDtypeStruct((TOTAL_M, D_EMB), jnp.float32),
        scratch_types=[
            pltpu.VMEM((SC_CHUNK,), jnp.int32),
            pltpu.VMEM((SC_CHUNK, D_EMB), jnp.float32),
            pltpu.SemaphoreType.DMA,
        ],
    )
    def gather_kernel(table_hbm, idx_hbm, out_hbm, idx_v, rows_v, sem):
        wid = lax.axis_index("s") * info.num_cores + lax.axis_index("c")
        base = wid * b_per_w
        for ci in range(n_chunks):
            b0 = base + ci * SC_CHUNK
            pltpu.sync_copy(idx_hbm.at[pl.ds(b0, SC_CHUNK)], idx_v)
            pltpu.async_copy(table_hbm.at[idx_v], rows_v, sem).wait()
            pltpu.sync_copy(rows_v, out_hbm.at[pl.ds(b0, SC_CHUNK)])

    return gather_kernel(table, idx_flat)


def kernel(inputs, embeddings):
    flat = inputs.reshape(TOTAL_M, D_EMB)
    idx2d, loss_sum = _tc_distance_argmin(flat, embeddings)
    out_flat = _sc_gather_rows(embeddings.T, idx2d.reshape(TOTAL_M))
    out = out_flat.reshape(inputs.shape)
    aux_loss = (1.0 + COMMIT_BETA) * (loss_sum[0, 0] / (TOTAL_M * D_EMB))
    return out, aux_loss
